# Initial kernel scaffold; baseline (speedup 1.0000x reference)
import jax
import jax.numpy as jnp
from jax.experimental import pallas as pl


def kernel(x, W_lin, b_lin, mem_w, la_w, gamma, beta):
    raise NotImplementedError("write your pallas kernel here")


# speedup vs baseline: 17.5510x; 17.5510x over previous
import jax
import jax.numpy as jnp
from jax.experimental import pallas as pl
from jax.experimental.pallas import tpu as pltpu

MEM = 50
LA = 5
EPS = 1e-5

LB = 256
HALO = 64
EXT = LB + 2 * HALO


def _build_kernel(mw_ref, lw_ref, a_ref):
    wm = jnp.sum(mw_ref[...], axis=1, keepdims=True)
    wf = jnp.sum(lw_ref[...], axis=1, keepdims=True)
    p = jax.lax.broadcasted_iota(jnp.int32, (LB, EXT), 0)
    q = jax.lax.broadcasted_iota(jnp.int32, (LB, EXT), 1)
    d = q - p - (HALO - MEM)
    a = jnp.where(d == MEM, 1.0, 0.0)
    for j in range(MEM):
        a = a + jnp.where(d == j, wm[j : j + 1, 0:1], 0.0)
    for k in range(LA):
        a = a + jnp.where(d == MEM + 1 + k, wf[k : k + 1, 0:1], 0.0)
    head = jnp.zeros((LB, EXT), jnp.float32)
    for j in range(MEM - 1):
        head = head + jnp.where(
            (q == HALO + j) & (p > j) & (p < MEM), wm[j : j + 1, 0:1], 0.0
        )
    a0 = jnp.where((p >= MEM) | (q - p >= HALO), a, 0.0) + head
    alast = jnp.where(q < HALO + LB, a, 0.0)
    a_ref[0] = a
    a_ref[1] = a0
    a_ref[2] = alast


def _linear_kernel(x_ref, w_ref, b_ref, o_ref):
    o_ref[0] = (
        jnp.dot(x_ref[0], w_ref[...], preferred_element_type=jnp.float32)
        + b_ref[...]
    )


def _fsmn_kernel(a_ref, lh_ref, cur_ref, rh_ref, g_ref, bt_ref, o_ref):
    i = pl.program_id(1)
    nlb = pl.num_programs(1)
    m = jnp.where(i == 0, a_ref[1], jnp.where(i == nlb - 1, a_ref[2], a_ref[0]))
    ext = jnp.concatenate([lh_ref[0], cur_ref[0], rh_ref[0]], axis=0)
    band = jnp.dot(m, ext, preferred_element_type=jnp.float32)
    mu = jnp.mean(band, axis=1, keepdims=True)
    xc = band - mu
    var = jnp.mean(xc * xc, axis=1, keepdims=True)
    y = xc * jax.lax.rsqrt(var + EPS)
    o_ref[0] = y * g_ref[...] + bt_ref[...]


def kernel(x, W_lin, b_lin, mem_w, la_w, gamma, beta):
    B, L, D = x.shape
    H = W_lin.shape[0]
    BM = 512
    wt = W_lin.T
    b2 = b_lin.reshape(1, H)
    g2 = gamma.reshape(1, H)
    bt2 = beta.reshape(1, H)

    a3 = pl.pallas_call(
        _build_kernel,
        out_shape=jax.ShapeDtypeStruct((3, LB, EXT), jnp.float32),
        name="dfsmn_build_band",
    )(mem_w, la_w)

    h = pl.pallas_call(
        _linear_kernel,
        grid=(B, L // BM),
        in_specs=[
            pl.BlockSpec((1, BM, D), lambda b, i: (b, i, 0)),
            pl.BlockSpec((D, H), lambda b, i: (0, 0)),
            pl.BlockSpec((1, H), lambda b, i: (0, 0)),
        ],
        out_specs=pl.BlockSpec((1, BM, H), lambda b, i: (b, i, 0)),
        out_shape=jax.ShapeDtypeStruct((B, L, H), jnp.float32),
        compiler_params=pltpu.CompilerParams(
            dimension_semantics=("parallel", "arbitrary"),
            vmem_limit_bytes=48 * 1024 * 1024,
        ),
        name="dfsmn_linear",
    )(x, wt, b2)

    nh = L // HALO
    out = pl.pallas_call(
        _fsmn_kernel,
        grid=(B, L // LB),
        in_specs=[
            pl.BlockSpec((3, LB, EXT), lambda b, i: (0, 0, 0)),
            pl.BlockSpec(
                (1, HALO, H),
                lambda b, i: (b, jnp.maximum(i * (LB // HALO) - 1, 0), 0),
            ),
            pl.BlockSpec((1, LB, H), lambda b, i: (b, i, 0)),
            pl.BlockSpec(
                (1, HALO, H),
                lambda b, i: (b, jnp.minimum((i + 1) * (LB // HALO), nh - 1), 0),
            ),
            pl.BlockSpec((1, H), lambda b, i: (0, 0)),
            pl.BlockSpec((1, H), lambda b, i: (0, 0)),
        ],
        out_specs=pl.BlockSpec((1, LB, H), lambda b, i: (b, i, 0)),
        out_shape=jax.ShapeDtypeStruct((B, L, H), jnp.float32),
        compiler_params=pltpu.CompilerParams(
            dimension_semantics=("parallel", "arbitrary"),
            vmem_limit_bytes=32 * 1024 * 1024,
        ),
        name="dfsmn_band_ln",
    )(a3, h, h, h, g2, bt2)
    return out


# speedup vs baseline: 27.1923x; 1.5493x over previous
import jax
import jax.numpy as jnp
from jax.experimental import pallas as pl
from jax.experimental.pallas import tpu as pltpu

MEM = 50
LA = 5
EPS = 1e-5

LB = 512
SB = 256
LH = 64
RH = 8
EXTW = SB + LH + RH


def _build_band(mw, lw):
    wm = jnp.sum(mw, axis=1, keepdims=True)
    wf = jnp.sum(lw, axis=1, keepdims=True)
    p = jax.lax.broadcasted_iota(jnp.int32, (SB, EXTW), 0)
    q = jax.lax.broadcasted_iota(jnp.int32, (SB, EXTW), 1)
    d = q - p - (LH - MEM)
    a = jnp.where(d == MEM, 1.0, 0.0)
    for j in range(MEM):
        a = a + jnp.where(d == j, wm[j : j + 1, 0:1], 0.0)
    for k in range(LA):
        a = a + jnp.where(d == MEM + 1 + k, wf[k : k + 1, 0:1], 0.0)
    head = jnp.zeros((SB, EXTW), jnp.float32)
    for j in range(MEM - 1):
        head = head + jnp.where(
            (q == LH + j) & (p > j) & (p < MEM), wm[j : j + 1, 0:1], 0.0
        )
    a0 = jnp.where((p >= MEM) | (q - p >= LH), a, 0.0) + head
    alast = jnp.where(q < LH + SB, a, 0.0)
    return a, a0, alast


def _fused_kernel(
    mw_ref, lw_ref, xl_ref, xc_ref, xr_ref, w_ref, b_ref, g_ref, bt_ref,
    o_ref, a_ref, h_ref,
):
    b = pl.program_id(0)
    i = pl.program_id(1)
    nlb = pl.num_programs(1)

    @pl.when((b == 0) & (i == 0))
    def _():
        a, a0, alast = _build_band(mw_ref[...], lw_ref[...])
        a_ref[0] = a
        a_ref[1] = a0
        a_ref[2] = alast

    x_ext = jnp.concatenate([xl_ref[0], xc_ref[0], xr_ref[0]], axis=0)
    h_ref[...] = (
        jax.lax.dot_general(
            x_ext, w_ref[...],
            dimension_numbers=(((1,), (1,)), ((), ())),
            preferred_element_type=jnp.float32,
        )
        + b_ref[...]
    )

    for k in range(LB // SB):
        m = a_ref[0]
        if k == 0:
            m = jnp.where(i == 0, a_ref[1], m)
        if k == LB // SB - 1:
            m = jnp.where(i == nlb - 1, a_ref[2], m)
        band = jnp.dot(
            m, h_ref[k * SB : k * SB + EXTW], preferred_element_type=jnp.float32
        )
        mu = jnp.mean(band, axis=1, keepdims=True)
        xc = band - mu
        var = jnp.mean(xc * xc, axis=1, keepdims=True)
        y = xc * jax.lax.rsqrt(var + EPS)
        o_ref[0, k * SB : (k + 1) * SB] = y * g_ref[...] + bt_ref[...]


def kernel(x, W_lin, b_lin, mem_w, la_w, gamma, beta):
    B, L, D = x.shape
    H = W_lin.shape[0]
    b2 = b_lin.reshape(1, H)
    g2 = gamma.reshape(1, H)
    bt2 = beta.reshape(1, H)
    nlh = L // LH
    nrh = L // RH

    return pl.pallas_call(
        _fused_kernel,
        grid=(B, L // LB),
        in_specs=[
            pl.BlockSpec((MEM, H), lambda b, i: (0, 0)),
            pl.BlockSpec((LA, H), lambda b, i: (0, 0)),
            pl.BlockSpec(
                (1, LH, D),
                lambda b, i: (b, jnp.maximum(i * (LB // LH) - 1, 0), 0),
            ),
            pl.BlockSpec((1, LB, D), lambda b, i: (b, i, 0)),
            pl.BlockSpec(
                (1, RH, D),
                lambda b, i: (b, jnp.minimum((i + 1) * (LB // RH), nrh - 1), 0),
            ),
            pl.BlockSpec((H, D), lambda b, i: (0, 0)),
            pl.BlockSpec((1, H), lambda b, i: (0, 0)),
            pl.BlockSpec((1, H), lambda b, i: (0, 0)),
            pl.BlockSpec((1, H), lambda b, i: (0, 0)),
        ],
        out_specs=pl.BlockSpec((1, LB, H), lambda b, i: (b, i, 0)),
        out_shape=jax.ShapeDtypeStruct((B, L, H), jnp.float32),
        scratch_shapes=[
            pltpu.VMEM((3, SB, EXTW), jnp.float32),
            pltpu.VMEM((LB + LH + RH, H), jnp.float32),
        ],
        compiler_params=pltpu.CompilerParams(
            dimension_semantics=("arbitrary", "arbitrary"),
            vmem_limit_bytes=50 * 1024 * 1024,
        ),
        name="dfsmn_fused",
    )(mem_w, la_w, x, x, x, W_lin, b2, g2, bt2)


# speedup vs baseline: 30.8621x; 1.1350x over previous
import jax
import jax.numpy as jnp
from jax.experimental import pallas as pl
from jax.experimental.pallas import tpu as pltpu

MEM = 50
LA = 5
EPS = 1e-5

LB = 512
SB = 256
LH = 64
RH = 8
EXTW = SB + LH + RH


def _build_band(mw, lw):
    wm = jnp.sum(mw, axis=1, keepdims=True)
    wf = jnp.sum(lw, axis=1, keepdims=True)
    p = jax.lax.broadcasted_iota(jnp.int32, (SB, EXTW), 0)
    q = jax.lax.broadcasted_iota(jnp.int32, (SB, EXTW), 1)
    d = q - p - (LH - MEM)
    a = jnp.where(d == MEM, 1.0, 0.0)
    for j in range(MEM):
        a = a + jnp.where(d == j, wm[j : j + 1, 0:1], 0.0)
    for k in range(LA):
        a = a + jnp.where(d == MEM + 1 + k, wf[k : k + 1, 0:1], 0.0)
    head = jnp.zeros((SB, EXTW), jnp.float32)
    for j in range(MEM - 1):
        head = head + jnp.where(
            (q == LH + j) & (p > j) & (p < MEM), wm[j : j + 1, 0:1], 0.0
        )
    a0 = jnp.where((p >= MEM) | (q - p >= LH), a, 0.0) + head
    alast = jnp.where(q < LH + SB, a, 0.0)
    return a, a0, alast


def _fused_kernel(
    mw_ref, lw_ref, xl_ref, xc_ref, xr_ref, w_ref, b_ref, g_ref, bt_ref,
    o_ref, a_ref, xb_ref,
):
    b = pl.program_id(0)
    i = pl.program_id(1)
    nlb = pl.num_programs(1)

    @pl.when((b == 0) & (i == 0))
    def _():
        a, a0, alast = _build_band(mw_ref[...], lw_ref[...])
        a_ref[0] = a
        a_ref[1] = a0
        a_ref[2] = alast

    x_ext = jnp.concatenate([xl_ref[0], xc_ref[0], xr_ref[0]], axis=0)

    for k in range(LB // SB):
        m = a_ref[0]
        if k == 0:
            m = jnp.where(i == 0, a_ref[1], m)
        if k == LB // SB - 1:
            m = jnp.where(i == nlb - 1, a_ref[2], m)
        xb_ref[...] = jnp.dot(
            m, x_ext[k * SB : k * SB + EXTW], preferred_element_type=jnp.float32
        )
        rs = jnp.sum(m, axis=1, keepdims=True)
        band = (
            jax.lax.dot_general(
                xb_ref[...], w_ref[...],
                dimension_numbers=(((1,), (1,)), ((), ())),
                preferred_element_type=jnp.float32,
            )
            + rs * b_ref[...]
        )
        mu = jnp.mean(band, axis=1, keepdims=True)
        xc = band - mu
        var = jnp.mean(xc * xc, axis=1, keepdims=True)
        y = xc * jax.lax.rsqrt(var + EPS)
        o_ref[0, k * SB : (k + 1) * SB] = y * g_ref[...] + bt_ref[...]


def kernel(x, W_lin, b_lin, mem_w, la_w, gamma, beta):
    B, L, D = x.shape
    H = W_lin.shape[0]
    b2 = b_lin.reshape(1, H)
    g2 = gamma.reshape(1, H)
    bt2 = beta.reshape(1, H)
    nlh = L // LH
    nrh = L // RH

    return pl.pallas_call(
        _fused_kernel,
        grid=(B, L // LB),
        in_specs=[
            pl.BlockSpec((MEM, H), lambda b, i: (0, 0)),
            pl.BlockSpec((LA, H), lambda b, i: (0, 0)),
            pl.BlockSpec(
                (1, LH, D),
                lambda b, i: (b, jnp.maximum(i * (LB // LH) - 1, 0), 0),
            ),
            pl.BlockSpec((1, LB, D), lambda b, i: (b, i, 0)),
            pl.BlockSpec(
                (1, RH, D),
                lambda b, i: (b, jnp.minimum((i + 1) * (LB // RH), nrh - 1), 0),
            ),
            pl.BlockSpec((H, D), lambda b, i: (0, 0)),
            pl.BlockSpec((1, H), lambda b, i: (0, 0)),
            pl.BlockSpec((1, H), lambda b, i: (0, 0)),
            pl.BlockSpec((1, H), lambda b, i: (0, 0)),
        ],
        out_specs=pl.BlockSpec((1, LB, H), lambda b, i: (b, i, 0)),
        out_shape=jax.ShapeDtypeStruct((B, L, H), jnp.float32),
        scratch_shapes=[
            pltpu.VMEM((3, SB, EXTW), jnp.float32),
            pltpu.VMEM((SB, D), jnp.float32),
        ],
        compiler_params=pltpu.CompilerParams(
            dimension_semantics=("arbitrary", "arbitrary"),
            vmem_limit_bytes=44 * 1024 * 1024,
        ),
        name="dfsmn_fused",
    )(mem_w, la_w, x, x, x, W_lin, b2, g2, bt2)
